# baseline (speedup 1.0000x reference)
import functools

import jax
import jax.numpy as jnp
import numpy as np
from jax import lax
from jax.experimental import pallas as pl
from jax.experimental.pallas import tpu as pltpu
from jax.experimental.pallas import tpu_sc as plsc

N = 10000
E = 160000
EMB = 1024
H = 3
CW = 128
NCH = (H * EMB) // CW
BM = 256

NEG_SLOPE = 0.2


def _pad_edges(src, dst, mask, n):
    ep = src.shape[0] + n
    ept = int(-(-ep // 1024) * 1024)
    loops = jnp.arange(n, dtype=jnp.int32)
    src = jnp.concatenate([src, loops, jnp.zeros((ept - ep,), jnp.int32)])
    dst = jnp.concatenate([dst, loops, jnp.zeros((ept - ep,), jnp.int32)])
    mask = jnp.concatenate([mask, jnp.ones((n,), jnp.float32),
                            jnp.zeros((ept - ep,), jnp.float32)])
    return src, dst, mask


def _mA_body(x_ref, w_ref, as_ref, ad_ref, *out_refs):
    h_refs = out_refs[:NCH]
    als_ref, ald_ref = out_refs[NCH], out_refs[NCH + 1]
    h = jnp.dot(x_ref[...], w_ref[...], preferred_element_type=jnp.float32)
    rows_s, rows_d = [], []
    for hh in range(H):
        sl = h[:, hh * EMB:(hh + 1) * EMB]
        rows_s.append(jnp.sum(sl * as_ref[hh:hh + 1, :], axis=1)[None, :])
        rows_d.append(jnp.sum(sl * ad_ref[hh:hh + 1, :], axis=1)[None, :])
    z = jnp.zeros((8 - H, x_ref.shape[0]), jnp.float32)
    als_ref[...] = jnp.concatenate(rows_s + [z], axis=0)
    ald_ref[...] = jnp.concatenate(rows_d + [z], axis=0)
    for c in range(NCH):
        h_refs[c][...] = h[:, c * CW:(c + 1) * CW]


def _mA(x, W, a_s, a_d):
    n, f = x.shape
    as_p = jnp.zeros((8, EMB), jnp.float32).at[:H].set(a_s)
    ad_p = jnp.zeros((8, EMB), jnp.float32).at[:H].set(a_d)
    grid = (n // BM,)
    out_shapes = ([jax.ShapeDtypeStruct((n, CW), jnp.float32)] * NCH
                  + [jax.ShapeDtypeStruct((8, n), jnp.float32)] * 2)
    out_specs = ([pl.BlockSpec((BM, CW), lambda m: (m, 0))] * NCH
                 + [pl.BlockSpec((8, BM), lambda m: (0, m))] * 2)
    res = pl.pallas_call(
        _mA_body,
        grid=grid,
        in_specs=[
            pl.BlockSpec((BM, f), lambda m: (m, 0)),
            pl.BlockSpec((f, H * EMB), lambda m: (0, 0)),
            pl.BlockSpec((8, EMB), lambda m: (0, 0)),
            pl.BlockSpec((8, EMB), lambda m: (0, 0)),
        ],
        out_specs=out_specs,
        out_shape=out_shapes,
    )(x, W, as_p, ad_p)
    return res[:NCH], res[NCH], res[NCH + 1]


def _mB_body(*refs):
    acc_refs = refs[:NCH]
    wh_ref, bc_ref, bh_ref, pw_ref = refs[NCH:NCH + 4]
    h2_ref, sc_ref = refs[NCH + 4], refs[NCH + 5]
    bm = acc_refs[0].shape[0]
    h2 = jnp.zeros((bm, EMB), jnp.float32)
    for c in range(NCH):
        xc = acc_refs[c][...] + bc_ref[c:c + 1, :]
        h2 = h2 + jnp.dot(xc, wh_ref[c * CW:(c + 1) * CW, :],
                          preferred_element_type=jnp.float32)
    h2 = h2 + bh_ref[0:1, :]
    h2_ref[...] = h2
    wnorm = jax.lax.rsqrt(jnp.sum(pw_ref[0:1, :] * pw_ref[0:1, :]))
    score = jnp.sum(h2 * pw_ref[0:1, :], axis=1) * wnorm
    z = jnp.zeros((7, bm), jnp.float32)
    sc_ref[...] = jnp.concatenate([score[None, :], z], axis=0)


def _mB(acc_chunks, Wh, bc, bh, pw):
    n = acc_chunks[0].shape[0]
    WhT = Wh.T
    bcp = bc.reshape(NCH, CW)
    bhp = jnp.zeros((8, EMB), jnp.float32).at[0].set(bh)
    pwp = jnp.zeros((8, EMB), jnp.float32).at[0].set(pw)
    grid = (n // BM,)
    h2, score = pl.pallas_call(
        _mB_body,
        grid=grid,
        in_specs=(
            [pl.BlockSpec((BM, CW), lambda m: (m, 0))] * NCH
            + [pl.BlockSpec((H * EMB, EMB), lambda m: (0, 0)),
               pl.BlockSpec((NCH, CW), lambda m: (0, 0)),
               pl.BlockSpec((8, EMB), lambda m: (0, 0)),
               pl.BlockSpec((8, EMB), lambda m: (0, 0))]
        ),
        out_specs=[pl.BlockSpec((BM, EMB), lambda m: (m, 0)),
                   pl.BlockSpec((8, BM), lambda m: (0, m))],
        out_shape=[jax.ShapeDtypeStruct((n, EMB), jnp.float32),
                   jax.ShapeDtypeStruct((8, n), jnp.float32)],
    )(*acc_chunks, WhT, bcp, bhp, pwp)
    return h2, score


def _readout_body(k, x_ref, out_ref):
    m = pl.program_id(0)
    rid = lax.broadcasted_iota(jnp.int32, (BM, EMB), 0) + m * BM
    valid = rid < k
    xb = x_ref[...]
    bmax = jnp.max(jnp.where(valid, xb, -1e30), axis=0)[None, :]
    bsum = (jnp.sum(jnp.where(valid, xb, 0.0), axis=0) / float(k))[None, :]
    blk = jnp.concatenate(
        [jnp.concatenate([bmax, bsum], axis=1),
         jnp.zeros((7, 2 * EMB), jnp.float32)], axis=0)

    @pl.when(m == 0)
    def _():
        out_ref[...] = blk

    @pl.when(m != 0)
    def _():
        prev = out_ref[...]
        mx = jnp.maximum(prev[0:1, :EMB], blk[0:1, :EMB])
        sm = prev[0:1, EMB:] + blk[0:1, EMB:]
        out_ref[...] = jnp.concatenate(
            [jnp.concatenate([mx, sm], axis=1), prev[1:]], axis=0)


def _readout(x, k):
    kpd = x.shape[0]
    return pl.pallas_call(
        functools.partial(_readout_body, k),
        grid=(kpd // BM,),
        in_specs=[pl.BlockSpec((BM, EMB), lambda m: (m, 0))],
        out_specs=pl.BlockSpec((8, 2 * EMB), lambda m: (0, 0)),
        out_shape=jax.ShapeDtypeStruct((8, 2 * EMB), jnp.float32),
    )(x)


def _mlp_body(r1_ref, r2_ref, r3_ref, w1_ref, b1_ref, w2_ref, b2_ref, out_ref):
    z = r1_ref[...] + r2_ref[...] + r3_ref[...]
    z = jnp.dot(z, w1_ref[...], preferred_element_type=jnp.float32) + b1_ref[0:1, :]
    z = jnp.maximum(z, 0.0)
    out_ref[...] = jnp.dot(z, w2_ref[...], preferred_element_type=jnp.float32) + b2_ref[...]


def _mlp(r1, r2, r3, Wl1, bl1, Wl2, bl2):
    W1T = Wl1.T
    W2Tp = jnp.zeros((EMB, 128), jnp.float32).at[:, :2].set(Wl2.T)
    b1p = jnp.zeros((8, EMB), jnp.float32).at[0].set(bl1)
    b2p = jnp.zeros((8, 128), jnp.float32).at[0, :2].set(bl2)
    out = pl.pallas_call(
        _mlp_body,
        in_specs=[
            pl.BlockSpec((8, 2 * EMB), lambda: (0, 0)),
            pl.BlockSpec((8, 2 * EMB), lambda: (0, 0)),
            pl.BlockSpec((8, 2 * EMB), lambda: (0, 0)),
            pl.BlockSpec((2 * EMB, EMB), lambda: (0, 0)),
            pl.BlockSpec((8, EMB), lambda: (0, 0)),
            pl.BlockSpec((EMB, 128), lambda: (0, 0)),
            pl.BlockSpec((8, 128), lambda: (0, 0)),
        ],
        out_specs=pl.BlockSpec((8, 128), lambda: (0, 0)),
        out_shape=jax.ShapeDtypeStruct((8, 128), jnp.float32),
    )(r1, r2, r3, W1T, b1p, W2Tp, b2p)
    return out[0:1, :2]


def _npad(n):
    return int(-(-n // 256) * 256)


_SC_MESH = dict(core_axis_name="c", subcore_axis_name="s")
_SC_PARAMS = pltpu.CompilerParams(needs_layout_passes=False)


def _edge_softmax_sc(als, ald, src, dst, mask, n):
    ep = src.shape[0]
    npd = _npad(n)
    ept = ep // 16
    ntile = npd // 16
    nb = ept // 16
    mesh = plsc.VectorSubcoreMesh(**_SC_MESH)

    @functools.partial(
        pl.kernel,
        out_type=[jax.ShapeDtypeStruct((ep,), jnp.float32)] * H,
        mesh=mesh,
        compiler_params=_SC_PARAMS,
        scratch_types=[
            pltpu.VMEM((npd,), jnp.float32),
            pltpu.VMEM((npd,), jnp.float32),
            pltpu.VMEM((npd,), jnp.float32),
            pltpu.VMEM((ept,), jnp.float32),
            pltpu.VMEM((ept,), jnp.int32),
            pltpu.VMEM((ept,), jnp.int32),
            pltpu.VMEM((ept,), jnp.float32),
            pltpu.VMEM((16, ntile), jnp.float32),
            pltpu.VMEM((ntile,), jnp.float32),
            pltpu.VMEM_SHARED((16, npd), jnp.float32),
            pltpu.VMEM_SHARED((npd,), jnp.float32),
        ],
    )
    def kern(als_h, ald_h, src_h, dst_h, mask_h, a0_h, a1_h, a2_h,
             als_t, ald_t, s_t, ex_t, srcv, dstv, maskv, red16, red,
             stage, merged):
        core = lax.axis_index("c")
        sub = lax.axis_index("s")
        base = sub * ept
        pltpu.sync_copy(src_h.at[pl.ds(base, ept)], srcv)
        pltpu.sync_copy(dst_h.at[pl.ds(base, ept)], dstv)
        pltpu.sync_copy(mask_h.at[pl.ds(base, ept)], maskv)
        alpha_outs = [a0_h, a1_h, a2_h]
        zero16 = jnp.zeros((16,), jnp.float32)
        for hh in range(H):
            ow = 0 if hh < 2 else 1

            @pl.when(core == ow)
            def _(hh=hh):
                a_h = alpha_outs[hh]
                pltpu.sync_copy(als_h.at[hh], als_t)
                pltpu.sync_copy(ald_h.at[hh], ald_t)

                def zb(i, _):
                    s_t[pl.ds(i * 16, 16)] = zero16
                    return 0
                lax.fori_loop(0, npd // 16, zb, 0)

                def p1(b, _):
                    sl = pl.ds(b * 16, 16)
                    d16 = dstv[sl]
                    e = (plsc.load_gather(als_t, [srcv[sl]])
                         + plsc.load_gather(ald_t, [d16]))
                    e = jnp.where(e >= 0, e, NEG_SLOPE * e)
                    exv = jnp.exp(e) * maskv[sl]
                    ex_t[sl] = exv
                    plsc.addupdate_scatter(s_t, [d16], exv)
                    return 0
                lax.fori_loop(0, nb, p1, 0)

                pltpu.sync_copy(s_t, stage.at[sub])
                plsc.subcore_barrier()
                pltpu.sync_copy(stage.at[:, pl.ds(sub * ntile, ntile)], red16)

                def p2(j, _):
                    sl = pl.ds(j * 16, 16)
                    acc = zero16
                    for r in range(16):
                        acc = acc + red16[r, sl]
                    red[sl] = acc
                    return 0
                lax.fori_loop(0, ntile // 16, p2, 0)
                pltpu.sync_copy(red, merged.at[pl.ds(sub * ntile, ntile)])
                plsc.subcore_barrier()
                pltpu.sync_copy(merged, s_t)
                plsc.subcore_barrier()

                def p3(b, _):
                    sl = pl.ds(b * 16, 16)
                    sg = plsc.load_gather(s_t, [dstv[sl]])
                    ex_t[sl] = ex_t[sl] / (sg + 1e-16)
                    return 0
                lax.fori_loop(0, nb, p3, 0)
                pltpu.sync_copy(ex_t, a_h.at[pl.ds(base, ept)])

    return list(kern(als, ald, src, dst, mask))


def _aggregate_sc(h_chunks, alphas, src, dst, npd):
    ep = src.shape[0]
    ept = ep // 16
    nb = ept // 16
    nr = npd // 16
    mesh = plsc.VectorSubcoreMesh(**_SC_MESH)
    nring = 4

    @functools.partial(
        pl.kernel,
        out_type=[jax.ShapeDtypeStruct((npd, CW), jnp.float32)] * NCH,
        mesh=mesh,
        compiler_params=_SC_PARAMS,
        scratch_types=(
            [pltpu.VMEM((ept,), jnp.int32)] * 2
            + [pltpu.VMEM((ept,), jnp.float32)]
            + [pltpu.VMEM((16, CW), jnp.float32)] * nring
            + [pltpu.VMEM((64, CW), jnp.float32)]
            + [pltpu.SemaphoreType.DMA] * nring
            + [pltpu.SemaphoreType.DMA] * nring
            + [pltpu.VMEM_SHARED((npd, CW), jnp.float32)]
        ),
    )
    def kern(*refs):
        h_refs = refs[:NCH]
        a_refs = refs[NCH:NCH + H]
        src_h, dst_h = refs[NCH + H], refs[NCH + H + 1]
        out_refs = refs[NCH + H + 2:2 * NCH + H + 2]
        scr = refs[2 * NCH + H + 2:]
        srcv, dstv, av = scr[0], scr[1], scr[2]
        bufs = scr[3:3 + nring]
        zb = scr[3 + nring]
        gsem = scr[4 + nring:4 + 2 * nring]
        ssem = scr[4 + 2 * nring:4 + 3 * nring]
        acc = scr[4 + 3 * nring]

        core = lax.axis_index("c")
        sub = lax.axis_index("s")
        base = sub * ept
        r0 = sub * nr
        pltpu.sync_copy(src_h.at[pl.ds(base, ept)], srcv)
        pltpu.sync_copy(dst_h.at[pl.ds(base, ept)], dstv)
        zero16 = jnp.zeros((16,), jnp.float32)
        for r in range(64):
            for j in range(CW // 16):
                zb[r, pl.ds(j * 16, 16)] = zero16

        def issue_gather(b, i, h_ref):
            s16 = srcv[pl.ds(b * 16, 16)]
            pltpu.async_copy(h_ref.at[s16], bufs[i], gsem[i])

        def chunk(c, h_ref, out_ref, need_alpha, a_ref):
            if need_alpha:
                pltpu.sync_copy(a_ref.at[pl.ds(base, ept)], av)

            def zloop(i, _):
                pltpu.sync_copy(zb, acc.at[pl.ds(r0 + i * 64, 64)])
                return 0
            lax.fori_loop(0, nr // 64, zloop, 0)
            plsc.subcore_barrier()

            for i in range(min(nring - 1, 3)):
                issue_gather(i, i, h_ref)

            def quad(q, _):
                for i in range(nring):
                    b = nring * q + i
                    ip = (i + nring - 1) % nring
                    @pl.when(b + nring - 1 < nb)
                    def _():
                        @pl.when(b + nring - 1 >= nring)
                        def _():
                            pltpu.make_async_copy(h_ref.at[pl.ds(0, 16)],
                                                  bufs[ip], ssem[ip]).wait()
                        issue_gather(b + nring - 1, ip, h_ref)
                    pltpu.make_async_copy(h_ref.at[pl.ds(0, 16)],
                                          bufs[i], gsem[i]).wait()

                    def row(r, _):
                        ar = plsc.load_gather(
                            av, [jnp.full((16,), b * 16 + r, jnp.int32)])

                        def col(j, _):
                            sl = pl.ds(j * 16, 16)
                            bufs[i][r, sl] = bufs[i][r, sl] * ar
                            return 0
                        lax.fori_loop(0, CW // 16, col, 0)
                        return 0
                    lax.fori_loop(0, 16, row, 0)
                    d16 = dstv[pl.ds(b * 16, 16)]
                    pltpu.async_copy(bufs[i], acc.at[d16], ssem[i], add=True)
                return 0
            lax.fori_loop(0, nb // nring, quad, 0)
            for i in range(nring):
                pltpu.make_async_copy(h_ref.at[pl.ds(0, 16)], bufs[i],
                                      ssem[i]).wait()
            plsc.subcore_barrier()
            pltpu.sync_copy(acc.at[pl.ds(r0, nr)], out_ref.at[pl.ds(r0, nr)])

        for ow in range(2):
            @pl.when(core == ow)
            def _(ow=ow):
                last_hh = -1
                for c in range(ow * (NCH // 2), (ow + 1) * (NCH // 2)):
                    hh = c // (EMB // CW)
                    chunk(c, h_refs[c], out_refs[c], hh != last_hh,
                          a_refs[hh])
                    last_hh = hh

    return list(kern(*h_chunks, *alphas, src, dst))


def _edge_softmax_jnp(als, ald, src, dst, mask, n):
    a_s = als[:H].T
    a_d = ald[:H].T
    e = a_s[src] + a_d[dst]
    e = jnp.where(e >= 0, e, NEG_SLOPE * e)
    ex = mask[:, None] * jnp.exp(e)
    s = jax.ops.segment_sum(ex, dst, num_segments=n)
    alpha = ex / (s[dst] + 1e-16)
    return [alpha[:, hh] for hh in range(H)]


def _aggregate_jnp(h_chunks, alphas, src, dst, n):
    out = []
    for c in range(NCH):
        a = alphas[c // (EMB // CW)]
        msg = h_chunks[c][src] * a[:, None]
        out.append(jax.ops.segment_sum(msg, dst, num_segments=n))
    return out


def _f32_key(x):
    u = lax.bitcast_convert_type(x, jnp.int32)
    return jnp.where(u >= 0, u, u ^ jnp.int32(0x7FFFFFFF))


def _select_jnp(score_row, n, k, kpd, src, dst, mask):
    npd = score_row.shape[0]
    key = _f32_key(score_row)
    key = jnp.where(jnp.arange(npd) < n, key, jnp.int32(-2**31))
    kth = jnp.sort(key)[npd - k]
    gt = key > kth
    tie = key == kth
    quota = k - jnp.sum(gt.astype(jnp.int32))
    tie_rank = jnp.cumsum(tie.astype(jnp.int32))
    sel = gt | (tie & (tie_rank <= quota))
    pos = jnp.cumsum(sel.astype(jnp.int32)) - 1
    idx = jnp.where(sel, pos, -1).astype(jnp.int32)
    perm = jnp.zeros((kpd,), jnp.int32).at[
        jnp.where(sel, pos, kpd)].set(jnp.arange(npd, dtype=jnp.int32), mode="drop")
    sm = idx[src]
    dm = idx[dst]
    nm = (mask > 0) & (sm >= 0) & (dm >= 0)
    return perm, jnp.maximum(sm, 0), jnp.maximum(dm, 0), nm.astype(jnp.float32)


def _gather_scale_jnp(h2, score_row, perm):
    sc = score_row[perm]
    t = jnp.exp(-2.0 * jnp.abs(sc))
    tanh = jnp.sign(sc) * (1.0 - t) / (1.0 + t)
    return h2[perm] * tanh[:, None]


def _layer(x, src_b, dst_b, mask_b, n, k, W, a_s, a_d, bc, Wh, bh, pw):
    npd = x.shape[0]
    kpd = _npad(k)
    src, dst, mask = _pad_edges(src_b, dst_b, mask_b, n)
    h_chunks, als, ald = _mA(x, W, a_s, a_d)
    alphas = _edge_softmax_sc(als, ald, src, dst, mask, n)
    acc_chunks = _aggregate_sc(h_chunks, alphas, src, dst, npd)
    h2, score = _mB(acc_chunks, Wh, bc, bh, pw)
    score_row = score[0]
    perm, src_n, dst_n, mask_n = _select_jnp(score_row, n, k, kpd,
                                             src_b, dst_b, mask_b)
    xn = _gather_scale_jnp(h2, score_row, perm)
    r = _readout(xn, k)
    return xn, src_n, dst_n, mask_n, r


def kernel(x, edge_attr, edge_index, batch_index, W1, as1, ad1, bc1, Wh1, bh1,
           pw1, W2, as2, ad2, bc2, Wh2, bh2, pw2, W3, as3, ad3, bc3, Wh3, bh3,
           pw3, Wl1, bl1, Wl2, bl2):
    src = edge_index[0]
    dst = edge_index[1]
    mask = jnp.ones((E,), jnp.float32)
    x = jnp.concatenate([x, jnp.zeros((_npad(N) - N, x.shape[1]), x.dtype)])
    h, src, dst, mask, r1 = _layer(x, src, dst, mask, N, 8000,
                                   W1, as1, ad1, bc1, Wh1, bh1, pw1)
    h, src, dst, mask, r2 = _layer(h, src, dst, mask, 8000, 4000,
                                   W2, as2, ad2, bc2, Wh2, bh2, pw2)
    h, src, dst, mask, r3 = _layer(h, src, dst, mask, 4000, 800,
                                   W3, as3, ad3, bc3, Wh3, bh3, pw3)
    return _mlp(r1, r2, r3, Wl1, bl1, Wl2, bl2)


# speedup vs baseline: 1.0001x; 1.0001x over previous
import functools

import jax
import jax.numpy as jnp
import numpy as np
from jax import lax
from jax.experimental import pallas as pl
from jax.experimental.pallas import tpu as pltpu
from jax.experimental.pallas import tpu_sc as plsc

N = 10000
E = 160000
EMB = 1024
H = 3
CW = 128
NCH = (H * EMB) // CW
BM = 256

NEG_SLOPE = 0.2


def _pad_edges(src, dst, mask, n):
    ep = src.shape[0] + n
    ept = int(-(-ep // 1024) * 1024)
    loops = jnp.arange(n, dtype=jnp.int32)
    src = jnp.concatenate([src, loops, jnp.zeros((ept - ep,), jnp.int32)])
    dst = jnp.concatenate([dst, loops, jnp.zeros((ept - ep,), jnp.int32)])
    mask = jnp.concatenate([mask, jnp.ones((n,), jnp.float32),
                            jnp.zeros((ept - ep,), jnp.float32)])
    return src, dst, mask


def _mA_body(x_ref, w_ref, as_ref, ad_ref, *out_refs):
    h_refs = out_refs[:NCH]
    als_ref, ald_ref = out_refs[NCH], out_refs[NCH + 1]
    h = jnp.dot(x_ref[...], w_ref[...], preferred_element_type=jnp.float32)
    rows_s, rows_d = [], []
    for hh in range(H):
        sl = h[:, hh * EMB:(hh + 1) * EMB]
        rows_s.append(jnp.sum(sl * as_ref[hh:hh + 1, :], axis=1)[None, :])
        rows_d.append(jnp.sum(sl * ad_ref[hh:hh + 1, :], axis=1)[None, :])
    z = jnp.zeros((8 - H, x_ref.shape[0]), jnp.float32)
    als_ref[...] = jnp.concatenate(rows_s + [z], axis=0)
    ald_ref[...] = jnp.concatenate(rows_d + [z], axis=0)
    for c in range(NCH):
        h_refs[c][...] = h[:, c * CW:(c + 1) * CW]


def _mA(x, W, a_s, a_d):
    n, f = x.shape
    as_p = jnp.zeros((8, EMB), jnp.float32).at[:H].set(a_s)
    ad_p = jnp.zeros((8, EMB), jnp.float32).at[:H].set(a_d)
    grid = (n // BM,)
    out_shapes = ([jax.ShapeDtypeStruct((n, CW), jnp.float32)] * NCH
                  + [jax.ShapeDtypeStruct((8, n), jnp.float32)] * 2)
    out_specs = ([pl.BlockSpec((BM, CW), lambda m: (m, 0))] * NCH
                 + [pl.BlockSpec((8, BM), lambda m: (0, m))] * 2)
    res = pl.pallas_call(
        _mA_body,
        grid=grid,
        in_specs=[
            pl.BlockSpec((BM, f), lambda m: (m, 0)),
            pl.BlockSpec((f, H * EMB), lambda m: (0, 0)),
            pl.BlockSpec((8, EMB), lambda m: (0, 0)),
            pl.BlockSpec((8, EMB), lambda m: (0, 0)),
        ],
        out_specs=out_specs,
        out_shape=out_shapes,
    )(x, W, as_p, ad_p)
    return res[:NCH], res[NCH], res[NCH + 1]


def _mB_body(*refs):
    acc_refs = refs[:NCH]
    wh_ref, bc_ref, bh_ref, pw_ref = refs[NCH:NCH + 4]
    h2_ref, sc_ref = refs[NCH + 4], refs[NCH + 5]
    bm = acc_refs[0].shape[0]
    h2 = jnp.zeros((bm, EMB), jnp.float32)
    for c in range(NCH):
        xc = acc_refs[c][...] + bc_ref[c:c + 1, :]
        h2 = h2 + jnp.dot(xc, wh_ref[c * CW:(c + 1) * CW, :],
                          preferred_element_type=jnp.float32)
    h2 = h2 + bh_ref[0:1, :]
    h2_ref[...] = h2
    wnorm = jax.lax.rsqrt(jnp.sum(pw_ref[0:1, :] * pw_ref[0:1, :]))
    score = jnp.sum(h2 * pw_ref[0:1, :], axis=1) * wnorm
    z = jnp.zeros((7, bm), jnp.float32)
    sc_ref[...] = jnp.concatenate([score[None, :], z], axis=0)


def _mB(acc_chunks, Wh, bc, bh, pw):
    n = acc_chunks[0].shape[0]
    WhT = Wh.T
    bcp = bc.reshape(NCH, CW)
    bhp = jnp.zeros((8, EMB), jnp.float32).at[0].set(bh)
    pwp = jnp.zeros((8, EMB), jnp.float32).at[0].set(pw)
    grid = (n // BM,)
    h2, score = pl.pallas_call(
        _mB_body,
        grid=grid,
        in_specs=(
            [pl.BlockSpec((BM, CW), lambda m: (m, 0))] * NCH
            + [pl.BlockSpec((H * EMB, EMB), lambda m: (0, 0)),
               pl.BlockSpec((NCH, CW), lambda m: (0, 0)),
               pl.BlockSpec((8, EMB), lambda m: (0, 0)),
               pl.BlockSpec((8, EMB), lambda m: (0, 0))]
        ),
        out_specs=[pl.BlockSpec((BM, EMB), lambda m: (m, 0)),
                   pl.BlockSpec((8, BM), lambda m: (0, m))],
        out_shape=[jax.ShapeDtypeStruct((n, EMB), jnp.float32),
                   jax.ShapeDtypeStruct((8, n), jnp.float32)],
    )(*acc_chunks, WhT, bcp, bhp, pwp)
    return h2, score


def _readout_body(k, x_ref, out_ref):
    m = pl.program_id(0)
    rid = lax.broadcasted_iota(jnp.int32, (BM, EMB), 0) + m * BM
    valid = rid < k
    xb = x_ref[...]
    bmax = jnp.max(jnp.where(valid, xb, -1e30), axis=0)[None, :]
    bsum = (jnp.sum(jnp.where(valid, xb, 0.0), axis=0) / float(k))[None, :]
    blk = jnp.concatenate(
        [jnp.concatenate([bmax, bsum], axis=1),
         jnp.zeros((7, 2 * EMB), jnp.float32)], axis=0)

    @pl.when(m == 0)
    def _():
        out_ref[...] = blk

    @pl.when(m != 0)
    def _():
        prev = out_ref[...]
        mx = jnp.maximum(prev[0:1, :EMB], blk[0:1, :EMB])
        sm = prev[0:1, EMB:] + blk[0:1, EMB:]
        out_ref[...] = jnp.concatenate(
            [jnp.concatenate([mx, sm], axis=1), prev[1:]], axis=0)


def _readout(x, k):
    kpd = x.shape[0]
    return pl.pallas_call(
        functools.partial(_readout_body, k),
        grid=(kpd // BM,),
        in_specs=[pl.BlockSpec((BM, EMB), lambda m: (m, 0))],
        out_specs=pl.BlockSpec((8, 2 * EMB), lambda m: (0, 0)),
        out_shape=jax.ShapeDtypeStruct((8, 2 * EMB), jnp.float32),
    )(x)


def _mlp_body(r1_ref, r2_ref, r3_ref, w1_ref, b1_ref, w2_ref, b2_ref, out_ref):
    z = r1_ref[...] + r2_ref[...] + r3_ref[...]
    z = jnp.dot(z, w1_ref[...], preferred_element_type=jnp.float32) + b1_ref[0:1, :]
    z = jnp.maximum(z, 0.0)
    out_ref[...] = jnp.dot(z, w2_ref[...], preferred_element_type=jnp.float32) + b2_ref[...]


def _mlp(r1, r2, r3, Wl1, bl1, Wl2, bl2):
    W1T = Wl1.T
    W2Tp = jnp.zeros((EMB, 128), jnp.float32).at[:, :2].set(Wl2.T)
    b1p = jnp.zeros((8, EMB), jnp.float32).at[0].set(bl1)
    b2p = jnp.zeros((8, 128), jnp.float32).at[0, :2].set(bl2)
    out = pl.pallas_call(
        _mlp_body,
        in_specs=[
            pl.BlockSpec((8, 2 * EMB), lambda: (0, 0)),
            pl.BlockSpec((8, 2 * EMB), lambda: (0, 0)),
            pl.BlockSpec((8, 2 * EMB), lambda: (0, 0)),
            pl.BlockSpec((2 * EMB, EMB), lambda: (0, 0)),
            pl.BlockSpec((8, EMB), lambda: (0, 0)),
            pl.BlockSpec((EMB, 128), lambda: (0, 0)),
            pl.BlockSpec((8, 128), lambda: (0, 0)),
        ],
        out_specs=pl.BlockSpec((8, 128), lambda: (0, 0)),
        out_shape=jax.ShapeDtypeStruct((8, 128), jnp.float32),
    )(r1, r2, r3, W1T, b1p, W2Tp, b2p)
    return out[0:1, :2]


def _npad(n):
    return int(-(-n // 256) * 256)


_SC_MESH = dict(core_axis_name="c", subcore_axis_name="s")
_SC_PARAMS = pltpu.CompilerParams(needs_layout_passes=False)


def _edge_softmax_sc(als, ald, src, dst, mask, n):
    ep = src.shape[0]
    npd = _npad(n)
    ept = ep // 16
    ntile = npd // 16
    nb = ept // 16
    mesh = plsc.VectorSubcoreMesh(**_SC_MESH)

    @functools.partial(
        pl.kernel,
        out_type=[jax.ShapeDtypeStruct((ep,), jnp.float32)] * H,
        mesh=mesh,
        compiler_params=_SC_PARAMS,
        scratch_types=[
            pltpu.VMEM((npd,), jnp.float32),
            pltpu.VMEM((npd,), jnp.float32),
            pltpu.VMEM((npd,), jnp.float32),
            pltpu.VMEM((ept,), jnp.float32),
            pltpu.VMEM((ept,), jnp.int32),
            pltpu.VMEM((ept,), jnp.int32),
            pltpu.VMEM((ept,), jnp.float32),
            pltpu.VMEM((16, ntile), jnp.float32),
            pltpu.VMEM((ntile,), jnp.float32),
            pltpu.VMEM_SHARED((16, npd), jnp.float32),
            pltpu.VMEM_SHARED((npd,), jnp.float32),
        ],
    )
    def kern(als_h, ald_h, src_h, dst_h, mask_h, a0_h, a1_h, a2_h,
             als_t, ald_t, s_t, ex_t, srcv, dstv, maskv, red16, red,
             stage, merged):
        core = lax.axis_index("c")
        sub = lax.axis_index("s")
        base = sub * ept
        pltpu.sync_copy(src_h.at[pl.ds(base, ept)], srcv)
        pltpu.sync_copy(dst_h.at[pl.ds(base, ept)], dstv)
        pltpu.sync_copy(mask_h.at[pl.ds(base, ept)], maskv)
        alpha_outs = [a0_h, a1_h, a2_h]
        zero16 = jnp.zeros((16,), jnp.float32)
        for hh in range(H):
            ow = 0 if hh < 2 else 1

            @pl.when(core == ow)
            def _(hh=hh):
                a_h = alpha_outs[hh]
                pltpu.sync_copy(als_h.at[hh], als_t)
                pltpu.sync_copy(ald_h.at[hh], ald_t)

                def zb(i, _):
                    s_t[pl.ds(i * 16, 16)] = zero16
                    return 0
                lax.fori_loop(0, npd // 16, zb, 0)

                def p1(b, _):
                    sl = pl.ds(b * 16, 16)
                    d16 = dstv[sl]
                    e = (plsc.load_gather(als_t, [srcv[sl]])
                         + plsc.load_gather(ald_t, [d16]))
                    e = jnp.where(e >= 0, e, NEG_SLOPE * e)
                    exv = jnp.exp(e) * maskv[sl]
                    ex_t[sl] = exv
                    plsc.addupdate_scatter(s_t, [d16], exv)
                    return 0
                lax.fori_loop(0, nb, p1, 0)

                pltpu.sync_copy(s_t, stage.at[sub])
                plsc.subcore_barrier()
                pltpu.sync_copy(stage.at[:, pl.ds(sub * ntile, ntile)], red16)

                def p2(j, _):
                    sl = pl.ds(j * 16, 16)
                    acc = zero16
                    for r in range(16):
                        acc = acc + red16[r, sl]
                    red[sl] = acc
                    return 0
                lax.fori_loop(0, ntile // 16, p2, 0)
                pltpu.sync_copy(red, merged.at[pl.ds(sub * ntile, ntile)])
                plsc.subcore_barrier()
                pltpu.sync_copy(merged, s_t)
                plsc.subcore_barrier()

                def p3(b, _):
                    sl = pl.ds(b * 16, 16)
                    sg = plsc.load_gather(s_t, [dstv[sl]])
                    ex_t[sl] = ex_t[sl] / (sg + 1e-16)
                    return 0
                lax.fori_loop(0, nb, p3, 0)
                pltpu.sync_copy(ex_t, a_h.at[pl.ds(base, ept)])

    return list(kern(als, ald, src, dst, mask))


def _aggregate_sc(h_chunks, alphas, src, dst, npd):
    ep = src.shape[0]
    ept = ep // 16
    nb = ept // 16
    nr = npd // 16
    mesh = plsc.VectorSubcoreMesh(**_SC_MESH)
    nring = 4

    @functools.partial(
        pl.kernel,
        out_type=[jax.ShapeDtypeStruct((npd, CW), jnp.float32)] * NCH,
        mesh=mesh,
        compiler_params=_SC_PARAMS,
        scratch_types=(
            [pltpu.VMEM((ept,), jnp.int32)] * 2
            + [pltpu.VMEM((ept,), jnp.float32)]
            + [pltpu.VMEM((16, CW), jnp.float32)] * nring
            + [pltpu.VMEM((64, CW), jnp.float32)]
            + [pltpu.SemaphoreType.DMA] * nring
            + [pltpu.SemaphoreType.DMA] * nring
            + [pltpu.VMEM_SHARED((npd, CW), jnp.float32)]
        ),
    )
    def kern(*refs):
        h_refs = refs[:NCH]
        a_refs = refs[NCH:NCH + H]
        src_h, dst_h = refs[NCH + H], refs[NCH + H + 1]
        out_refs = refs[NCH + H + 2:2 * NCH + H + 2]
        scr = refs[2 * NCH + H + 2:]
        srcv, dstv, av = scr[0], scr[1], scr[2]
        bufs = scr[3:3 + nring]
        zb = scr[3 + nring]
        gsem = scr[4 + nring:4 + 2 * nring]
        ssem = scr[4 + 2 * nring:4 + 3 * nring]
        acc = scr[4 + 3 * nring]

        core = lax.axis_index("c")
        sub = lax.axis_index("s")
        base = sub * ept
        r0 = sub * nr
        pltpu.sync_copy(src_h.at[pl.ds(base, ept)], srcv)
        pltpu.sync_copy(dst_h.at[pl.ds(base, ept)], dstv)
        zero16 = jnp.zeros((16,), jnp.float32)
        for r in range(64):
            for j in range(CW // 16):
                zb[r, pl.ds(j * 16, 16)] = zero16

        def issue_gather(b, i, h_ref):
            s16 = srcv[pl.ds(b * 16, 16)]
            pltpu.async_copy(h_ref.at[s16], bufs[i], gsem[i])

        def chunk(c, h_ref, out_ref, need_alpha, a_ref):
            if need_alpha:
                pltpu.sync_copy(a_ref.at[pl.ds(base, ept)], av)

            def zloop(i, _):
                pltpu.sync_copy(zb, acc.at[pl.ds(r0 + i * 64, 64)])
                return 0
            lax.fori_loop(0, nr // 64, zloop, 0)
            plsc.subcore_barrier()

            for i in range(min(nring - 1, 3)):
                issue_gather(i, i, h_ref)

            def quad(q, _):
                for i in range(nring):
                    b = nring * q + i
                    ip = (i + nring - 1) % nring
                    @pl.when(b + nring - 1 < nb)
                    def _():
                        @pl.when(b + nring - 1 >= nring)
                        def _():
                            pltpu.make_async_copy(h_ref.at[pl.ds(0, 16)],
                                                  bufs[ip], ssem[ip]).wait()
                        issue_gather(b + nring - 1, ip, h_ref)
                    pltpu.make_async_copy(h_ref.at[pl.ds(0, 16)],
                                          bufs[i], gsem[i]).wait()

                    def row(r, _):
                        ar = plsc.load_gather(
                            av, [jnp.full((16,), b * 16 + r, jnp.int32)])
                        for j in range(CW // 16):
                            sl = pl.ds(j * 16, 16)
                            bufs[i][r, sl] = bufs[i][r, sl] * ar
                        return 0
                    lax.fori_loop(0, 16, row, 0)
                    d16 = dstv[pl.ds(b * 16, 16)]
                    pltpu.async_copy(bufs[i], acc.at[d16], ssem[i], add=True)
                return 0
            lax.fori_loop(0, nb // nring, quad, 0)
            for i in range(nring):
                pltpu.make_async_copy(h_ref.at[pl.ds(0, 16)], bufs[i],
                                      ssem[i]).wait()
            plsc.subcore_barrier()
            pltpu.sync_copy(acc.at[pl.ds(r0, nr)], out_ref.at[pl.ds(r0, nr)])

        for ow in range(2):
            @pl.when(core == ow)
            def _(ow=ow):
                last_hh = -1
                for c in range(ow * (NCH // 2), (ow + 1) * (NCH // 2)):
                    hh = c // (EMB // CW)
                    chunk(c, h_refs[c], out_refs[c], hh != last_hh,
                          a_refs[hh])
                    last_hh = hh

    return list(kern(*h_chunks, *alphas, src, dst))


def _edge_softmax_jnp(als, ald, src, dst, mask, n):
    a_s = als[:H].T
    a_d = ald[:H].T
    e = a_s[src] + a_d[dst]
    e = jnp.where(e >= 0, e, NEG_SLOPE * e)
    ex = mask[:, None] * jnp.exp(e)
    s = jax.ops.segment_sum(ex, dst, num_segments=n)
    alpha = ex / (s[dst] + 1e-16)
    return [alpha[:, hh] for hh in range(H)]


def _aggregate_jnp(h_chunks, alphas, src, dst, n):
    out = []
    for c in range(NCH):
        a = alphas[c // (EMB // CW)]
        msg = h_chunks[c][src] * a[:, None]
        out.append(jax.ops.segment_sum(msg, dst, num_segments=n))
    return out


def _f32_key(x):
    u = lax.bitcast_convert_type(x, jnp.int32)
    return jnp.where(u >= 0, u, u ^ jnp.int32(0x7FFFFFFF))


def _select_jnp(score_row, n, k, kpd, src, dst, mask):
    npd = score_row.shape[0]
    key = _f32_key(score_row)
    key = jnp.where(jnp.arange(npd) < n, key, jnp.int32(-2**31))
    kth = jnp.sort(key)[npd - k]
    gt = key > kth
    tie = key == kth
    quota = k - jnp.sum(gt.astype(jnp.int32))
    tie_rank = jnp.cumsum(tie.astype(jnp.int32))
    sel = gt | (tie & (tie_rank <= quota))
    pos = jnp.cumsum(sel.astype(jnp.int32)) - 1
    idx = jnp.where(sel, pos, -1).astype(jnp.int32)
    perm = jnp.zeros((kpd,), jnp.int32).at[
        jnp.where(sel, pos, kpd)].set(jnp.arange(npd, dtype=jnp.int32), mode="drop")
    sm = idx[src]
    dm = idx[dst]
    nm = (mask > 0) & (sm >= 0) & (dm >= 0)
    return perm, jnp.maximum(sm, 0), jnp.maximum(dm, 0), nm.astype(jnp.float32)


def _gather_scale_jnp(h2, score_row, perm):
    sc = score_row[perm]
    t = jnp.exp(-2.0 * jnp.abs(sc))
    tanh = jnp.sign(sc) * (1.0 - t) / (1.0 + t)
    return h2[perm] * tanh[:, None]


def _layer(x, src_b, dst_b, mask_b, n, k, W, a_s, a_d, bc, Wh, bh, pw):
    npd = x.shape[0]
    kpd = _npad(k)
    src, dst, mask = _pad_edges(src_b, dst_b, mask_b, n)
    h_chunks, als, ald = _mA(x, W, a_s, a_d)
    alphas = _edge_softmax_sc(als, ald, src, dst, mask, n)
    acc_chunks = _aggregate_sc(h_chunks, alphas, src, dst, npd)
    h2, score = _mB(acc_chunks, Wh, bc, bh, pw)
    score_row = score[0]
    perm, src_n, dst_n, mask_n = _select_jnp(score_row, n, k, kpd,
                                             src_b, dst_b, mask_b)
    xn = _gather_scale_jnp(h2, score_row, perm)
    r = _readout(xn, k)
    return xn, src_n, dst_n, mask_n, r


def kernel(x, edge_attr, edge_index, batch_index, W1, as1, ad1, bc1, Wh1, bh1,
           pw1, W2, as2, ad2, bc2, Wh2, bh2, pw2, W3, as3, ad3, bc3, Wh3, bh3,
           pw3, Wl1, bl1, Wl2, bl2):
    src = edge_index[0]
    dst = edge_index[1]
    mask = jnp.ones((E,), jnp.float32)
    x = jnp.concatenate([x, jnp.zeros((_npad(N) - N, x.shape[1]), x.dtype)])
    h, src, dst, mask, r1 = _layer(x, src, dst, mask, N, 8000,
                                   W1, as1, ad1, bc1, Wh1, bh1, pw1)
    h, src, dst, mask, r2 = _layer(h, src, dst, mask, 8000, 4000,
                                   W2, as2, ad2, bc2, Wh2, bh2, pw2)
    h, src, dst, mask, r3 = _layer(h, src, dst, mask, 4000, 800,
                                   W3, as3, ad3, bc3, Wh3, bh3, pw3)
    return _mlp(r1, r2, r3, Wl1, bl1, Wl2, bl2)


# speedup vs baseline: 5.3283x; 5.3276x over previous
import functools

import jax
import jax.numpy as jnp
import numpy as np
from jax import lax
from jax.experimental import pallas as pl
from jax.experimental.pallas import tpu as pltpu
from jax.experimental.pallas import tpu_sc as plsc

N = 10000
E = 160000
EMB = 1024
H = 3
CW = 128
NCH = (H * EMB) // CW
BM = 256

NEG_SLOPE = 0.2


def _pad_edges(src, dst, mask, n):
    ep = src.shape[0] + n
    ept = int(-(-ep // 1024) * 1024)
    loops = jnp.arange(n, dtype=jnp.int32)
    pad_idx = jnp.arange(ept - ep, dtype=jnp.int32) % n
    src = jnp.concatenate([src, loops, pad_idx])
    dst = jnp.concatenate([dst, loops, pad_idx])
    mask = jnp.concatenate([mask, jnp.ones((n,), jnp.float32),
                            jnp.zeros((ept - ep,), jnp.float32)])
    return src, dst, mask


def _mA_body(x_ref, w_ref, as_ref, ad_ref, *out_refs):
    h_refs = out_refs[:NCH]
    als_ref, ald_ref = out_refs[NCH], out_refs[NCH + 1]
    h = jnp.dot(x_ref[...], w_ref[...], preferred_element_type=jnp.float32)
    rows_s, rows_d = [], []
    for hh in range(H):
        sl = h[:, hh * EMB:(hh + 1) * EMB]
        rows_s.append(jnp.sum(sl * as_ref[hh:hh + 1, :], axis=1)[None, :])
        rows_d.append(jnp.sum(sl * ad_ref[hh:hh + 1, :], axis=1)[None, :])
    z = jnp.zeros((8 - H, x_ref.shape[0]), jnp.float32)
    als_ref[...] = jnp.concatenate(rows_s + [z], axis=0)
    ald_ref[...] = jnp.concatenate(rows_d + [z], axis=0)
    for c in range(NCH):
        h_refs[c][...] = h[:, c * CW:(c + 1) * CW]


def _mA(x, W, a_s, a_d):
    n, f = x.shape
    as_p = jnp.zeros((8, EMB), jnp.float32).at[:H].set(a_s)
    ad_p = jnp.zeros((8, EMB), jnp.float32).at[:H].set(a_d)
    grid = (n // BM,)
    out_shapes = ([jax.ShapeDtypeStruct((n, CW), jnp.float32)] * NCH
                  + [jax.ShapeDtypeStruct((8, n), jnp.float32)] * 2)
    out_specs = ([pl.BlockSpec((BM, CW), lambda m: (m, 0))] * NCH
                 + [pl.BlockSpec((8, BM), lambda m: (0, m))] * 2)
    res = pl.pallas_call(
        _mA_body,
        grid=grid,
        in_specs=[
            pl.BlockSpec((BM, f), lambda m: (m, 0)),
            pl.BlockSpec((f, H * EMB), lambda m: (0, 0)),
            pl.BlockSpec((8, EMB), lambda m: (0, 0)),
            pl.BlockSpec((8, EMB), lambda m: (0, 0)),
        ],
        out_specs=out_specs,
        out_shape=out_shapes,
    )(x, W, as_p, ad_p)
    return res[:NCH], res[NCH], res[NCH + 1]


def _mB_body(*refs):
    acc_refs = refs[:NCH]
    wh_ref, bc_ref, bh_ref, pw_ref = refs[NCH:NCH + 4]
    h2_ref, sc_ref = refs[NCH + 4], refs[NCH + 5]
    bm = acc_refs[0].shape[0]
    h2 = jnp.zeros((bm, EMB), jnp.float32)
    for c in range(NCH):
        xc = acc_refs[c][...] + bc_ref[c:c + 1, :]
        h2 = h2 + jnp.dot(xc, wh_ref[c * CW:(c + 1) * CW, :],
                          preferred_element_type=jnp.float32)
    h2 = h2 + bh_ref[0:1, :]
    h2_ref[...] = h2
    wnorm = jax.lax.rsqrt(jnp.sum(pw_ref[0:1, :] * pw_ref[0:1, :]))
    score = jnp.sum(h2 * pw_ref[0:1, :], axis=1) * wnorm
    z = jnp.zeros((7, bm), jnp.float32)
    sc_ref[...] = jnp.concatenate([score[None, :], z], axis=0)


def _mB(acc_chunks, Wh, bc, bh, pw):
    n = acc_chunks[0].shape[0]
    WhT = Wh.T
    bcp = bc.reshape(NCH, CW)
    bhp = jnp.zeros((8, EMB), jnp.float32).at[0].set(bh)
    pwp = jnp.zeros((8, EMB), jnp.float32).at[0].set(pw)
    grid = (n // BM,)
    h2, score = pl.pallas_call(
        _mB_body,
        grid=grid,
        in_specs=(
            [pl.BlockSpec((BM, CW), lambda m: (m, 0))] * NCH
            + [pl.BlockSpec((H * EMB, EMB), lambda m: (0, 0)),
               pl.BlockSpec((NCH, CW), lambda m: (0, 0)),
               pl.BlockSpec((8, EMB), lambda m: (0, 0)),
               pl.BlockSpec((8, EMB), lambda m: (0, 0))]
        ),
        out_specs=[pl.BlockSpec((BM, EMB), lambda m: (m, 0)),
                   pl.BlockSpec((8, BM), lambda m: (0, m))],
        out_shape=[jax.ShapeDtypeStruct((n, EMB), jnp.float32),
                   jax.ShapeDtypeStruct((8, n), jnp.float32)],
    )(*acc_chunks, WhT, bcp, bhp, pwp)
    return h2, score


def _readout_body(k, x_ref, out_ref):
    m = pl.program_id(0)
    rid = lax.broadcasted_iota(jnp.int32, (BM, EMB), 0) + m * BM
    valid = rid < k
    xb = x_ref[...]
    bmax = jnp.max(jnp.where(valid, xb, -1e30), axis=0)[None, :]
    bsum = (jnp.sum(jnp.where(valid, xb, 0.0), axis=0) / float(k))[None, :]
    blk = jnp.concatenate(
        [jnp.concatenate([bmax, bsum], axis=1),
         jnp.zeros((7, 2 * EMB), jnp.float32)], axis=0)

    @pl.when(m == 0)
    def _():
        out_ref[...] = blk

    @pl.when(m != 0)
    def _():
        prev = out_ref[...]
        mx = jnp.maximum(prev[0:1, :EMB], blk[0:1, :EMB])
        sm = prev[0:1, EMB:] + blk[0:1, EMB:]
        out_ref[...] = jnp.concatenate(
            [jnp.concatenate([mx, sm], axis=1), prev[1:]], axis=0)


def _readout(x, k):
    kpd = x.shape[0]
    return pl.pallas_call(
        functools.partial(_readout_body, k),
        grid=(kpd // BM,),
        in_specs=[pl.BlockSpec((BM, EMB), lambda m: (m, 0))],
        out_specs=pl.BlockSpec((8, 2 * EMB), lambda m: (0, 0)),
        out_shape=jax.ShapeDtypeStruct((8, 2 * EMB), jnp.float32),
    )(x)


def _mlp_body(r1_ref, r2_ref, r3_ref, w1_ref, b1_ref, w2_ref, b2_ref, out_ref):
    z = r1_ref[...] + r2_ref[...] + r3_ref[...]
    z = jnp.dot(z, w1_ref[...], preferred_element_type=jnp.float32) + b1_ref[0:1, :]
    z = jnp.maximum(z, 0.0)
    out_ref[...] = jnp.dot(z, w2_ref[...], preferred_element_type=jnp.float32) + b2_ref[...]


def _mlp(r1, r2, r3, Wl1, bl1, Wl2, bl2):
    W1T = Wl1.T
    W2Tp = jnp.zeros((EMB, 128), jnp.float32).at[:, :2].set(Wl2.T)
    b1p = jnp.zeros((8, EMB), jnp.float32).at[0].set(bl1)
    b2p = jnp.zeros((8, 128), jnp.float32).at[0, :2].set(bl2)
    out = pl.pallas_call(
        _mlp_body,
        in_specs=[
            pl.BlockSpec((8, 2 * EMB), lambda: (0, 0)),
            pl.BlockSpec((8, 2 * EMB), lambda: (0, 0)),
            pl.BlockSpec((8, 2 * EMB), lambda: (0, 0)),
            pl.BlockSpec((2 * EMB, EMB), lambda: (0, 0)),
            pl.BlockSpec((8, EMB), lambda: (0, 0)),
            pl.BlockSpec((EMB, 128), lambda: (0, 0)),
            pl.BlockSpec((8, 128), lambda: (0, 0)),
        ],
        out_specs=pl.BlockSpec((8, 128), lambda: (0, 0)),
        out_shape=jax.ShapeDtypeStruct((8, 128), jnp.float32),
    )(r1, r2, r3, W1T, b1p, W2Tp, b2p)
    return out[0:1, :2]


def _npad(n):
    return int(-(-n // 256) * 256)


_SC_MESH = dict(core_axis_name="c", subcore_axis_name="s")
_SC_PARAMS = pltpu.CompilerParams(needs_layout_passes=False)


def _edge_softmax_sc(als, ald, src, dst, mask, n):
    ep = src.shape[0]
    npd = _npad(n)
    ept = ep // 16
    ntile = npd // 16
    nb = ept // 16
    mesh = plsc.VectorSubcoreMesh(**_SC_MESH)

    @functools.partial(
        pl.kernel,
        out_type=[jax.ShapeDtypeStruct((ep,), jnp.float32)] * H,
        mesh=mesh,
        compiler_params=_SC_PARAMS,
        scratch_types=[
            pltpu.VMEM((npd,), jnp.float32),
            pltpu.VMEM((npd,), jnp.float32),
            pltpu.VMEM((npd,), jnp.float32),
            pltpu.VMEM((ept,), jnp.float32),
            pltpu.VMEM((ept,), jnp.int32),
            pltpu.VMEM((ept,), jnp.int32),
            pltpu.VMEM((ept,), jnp.float32),
            pltpu.VMEM((16, ntile), jnp.float32),
            pltpu.VMEM((ntile,), jnp.float32),
            pltpu.VMEM_SHARED((16, npd), jnp.float32),
            pltpu.VMEM_SHARED((npd,), jnp.float32),
        ],
    )
    def kern(als_h, ald_h, src_h, dst_h, mask_h, a0_h, a1_h, a2_h,
             als_t, ald_t, s_t, ex_t, srcv, dstv, maskv, red16, red,
             stage, merged):
        core = lax.axis_index("c")
        sub = lax.axis_index("s")
        base = sub * ept
        pltpu.sync_copy(src_h.at[pl.ds(base, ept)], srcv)
        pltpu.sync_copy(dst_h.at[pl.ds(base, ept)], dstv)
        pltpu.sync_copy(mask_h.at[pl.ds(base, ept)], maskv)
        alpha_outs = [a0_h, a1_h, a2_h]
        zero16 = jnp.zeros((16,), jnp.float32)
        for hh in range(H):
            ow = 0 if hh < 2 else 1

            @pl.when(core == ow)
            def _(hh=hh):
                a_h = alpha_outs[hh]
                pltpu.sync_copy(als_h.at[hh], als_t)
                pltpu.sync_copy(ald_h.at[hh], ald_t)

                def zb(i, _):
                    s_t[pl.ds(i * 16, 16)] = zero16
                    return 0
                lax.fori_loop(0, npd // 16, zb, 0)

                def p1(b, _):
                    sl = pl.ds(b * 16, 16)
                    d16 = dstv[sl]
                    e = (plsc.load_gather(als_t, [srcv[sl]])
                         + plsc.load_gather(ald_t, [d16]))
                    e = jnp.where(e >= 0, e, NEG_SLOPE * e)
                    exv = jnp.exp(e) * maskv[sl]
                    ex_t[sl] = exv
                    plsc.addupdate_scatter(s_t, [d16], exv)
                    return 0
                lax.fori_loop(0, nb, p1, 0)

                pltpu.sync_copy(s_t, stage.at[sub])
                plsc.subcore_barrier()
                pltpu.sync_copy(stage.at[:, pl.ds(sub * ntile, ntile)], red16)

                def p2(j, _):
                    sl = pl.ds(j * 16, 16)
                    acc = zero16
                    for r in range(16):
                        acc = acc + red16[r, sl]
                    red[sl] = acc
                    return 0
                lax.fori_loop(0, ntile // 16, p2, 0)
                pltpu.sync_copy(red, merged.at[pl.ds(sub * ntile, ntile)])
                plsc.subcore_barrier()
                pltpu.sync_copy(merged, s_t)
                plsc.subcore_barrier()

                def p3(b, _):
                    sl = pl.ds(b * 16, 16)
                    sg = plsc.load_gather(s_t, [dstv[sl]])
                    ex_t[sl] = ex_t[sl] / (sg + 1e-16)
                    return 0
                lax.fori_loop(0, nb, p3, 0)
                pltpu.sync_copy(ex_t, a_h.at[pl.ds(base, ept)])

    return list(kern(als, ald, src, dst, mask))


def _aggregate_sc(h_chunks, alphas, src, dst, npd):
    ep = src.shape[0]
    ept = ep // 16
    nb = ept // 16
    nr = npd // 16
    mesh = plsc.VectorSubcoreMesh(**_SC_MESH)
    nring = 4

    @functools.partial(
        pl.kernel,
        out_type=[jax.ShapeDtypeStruct((npd, CW), jnp.float32)] * NCH,
        mesh=mesh,
        compiler_params=_SC_PARAMS,
        scratch_types=(
            [pltpu.VMEM((ept,), jnp.int32)] * 2
            + [pltpu.VMEM((ept,), jnp.float32)]
            + [pltpu.VMEM((16, CW), jnp.float32)] * nring
            + [pltpu.VMEM((64, CW), jnp.float32)]
            + [pltpu.SemaphoreType.DMA] * nring
            + [pltpu.SemaphoreType.DMA] * nring
            + [pltpu.VMEM_SHARED((npd, CW), jnp.float32)]
        ),
    )
    def kern(*refs):
        h_refs = refs[:NCH]
        a_refs = refs[NCH:NCH + H]
        src_h, dst_h = refs[NCH + H], refs[NCH + H + 1]
        out_refs = refs[NCH + H + 2:2 * NCH + H + 2]
        scr = refs[2 * NCH + H + 2:]
        srcv, dstv, av = scr[0], scr[1], scr[2]
        bufs = scr[3:3 + nring]
        zb = scr[3 + nring]
        gsem = scr[4 + nring:4 + 2 * nring]
        ssem = scr[4 + 2 * nring:4 + 3 * nring]
        acc = scr[4 + 3 * nring]

        core = lax.axis_index("c")
        sub = lax.axis_index("s")
        base = sub * ept
        r0 = sub * nr
        pltpu.sync_copy(src_h.at[pl.ds(base, ept)], srcv)
        pltpu.sync_copy(dst_h.at[pl.ds(base, ept)], dstv)
        zero16 = jnp.zeros((16,), jnp.float32)
        for r in range(64):
            for j in range(CW // 16):
                zb[r, pl.ds(j * 16, 16)] = zero16

        def issue_gather(b, i, h_ref):
            s16 = srcv[pl.ds(b * 16, 16)]
            pltpu.async_copy(h_ref.at[s16], bufs[i], gsem[i])

        def chunk(c, h_ref, out_ref, need_alpha, a_ref):
            if need_alpha:
                pltpu.sync_copy(a_ref.at[pl.ds(base, ept)], av)

            def zloop(i, _):
                pltpu.sync_copy(zb, acc.at[pl.ds(r0 + i * 64, 64)])
                return 0
            lax.fori_loop(0, nr // 64, zloop, 0)
            plsc.subcore_barrier()

            for i in range(min(nring - 1, 3)):
                issue_gather(i, i, h_ref)

            def quad(q, _):
                for i in range(nring):
                    b = nring * q + i
                    ip = (i + nring - 1) % nring
                    @pl.when(b + nring - 1 < nb)
                    def _():
                        @pl.when(b + nring - 1 >= nring)
                        def _():
                            pltpu.make_async_copy(h_ref.at[pl.ds(0, 16)],
                                                  bufs[ip], ssem[ip]).wait()
                        issue_gather(b + nring - 1, ip, h_ref)
                    pltpu.make_async_copy(h_ref.at[pl.ds(0, 16)],
                                          bufs[i], gsem[i]).wait()

                    def row(r, _):
                        ar = plsc.load_gather(
                            av, [jnp.full((16,), b * 16 + r, jnp.int32)])
                        for j in range(CW // 16):
                            sl = pl.ds(j * 16, 16)
                            bufs[i][r, sl] = bufs[i][r, sl] * ar
                        return 0
                    lax.fori_loop(0, 16, row, 0)
                    d16 = dstv[pl.ds(b * 16, 16)]
                    pltpu.async_copy(bufs[i], acc.at[d16], ssem[i], add=True)
                return 0
            lax.fori_loop(0, nb // nring, quad, 0)
            for i in range(nring):
                pltpu.make_async_copy(h_ref.at[pl.ds(0, 16)], bufs[i],
                                      ssem[i]).wait()
            plsc.subcore_barrier()
            pltpu.sync_copy(acc.at[pl.ds(r0, nr)], out_ref.at[pl.ds(r0, nr)])

        for ow in range(2):
            @pl.when(core == ow)
            def _(ow=ow):
                last_hh = -1
                for c in range(ow * (NCH // 2), (ow + 1) * (NCH // 2)):
                    hh = c // (EMB // CW)
                    chunk(c, h_refs[c], out_refs[c], hh != last_hh,
                          a_refs[hh])
                    last_hh = hh

    return list(kern(*h_chunks, *alphas, src, dst))


def _edge_softmax_jnp(als, ald, src, dst, mask, n):
    a_s = als[:H].T
    a_d = ald[:H].T
    e = a_s[src] + a_d[dst]
    e = jnp.where(e >= 0, e, NEG_SLOPE * e)
    ex = mask[:, None] * jnp.exp(e)
    s = jax.ops.segment_sum(ex, dst, num_segments=n)
    alpha = ex / (s[dst] + 1e-16)
    return [alpha[:, hh] for hh in range(H)]


def _aggregate_jnp(h_chunks, alphas, src, dst, n):
    out = []
    for c in range(NCH):
        a = alphas[c // (EMB // CW)]
        msg = h_chunks[c][src] * a[:, None]
        out.append(jax.ops.segment_sum(msg, dst, num_segments=n))
    return out


def _f32_key(x):
    u = lax.bitcast_convert_type(x, jnp.int32)
    return jnp.where(u >= 0, u, u ^ jnp.int32(0x7FFFFFFF))


def _select_jnp(score_row, n, k, kpd, src, dst, mask):
    npd = score_row.shape[0]
    key = _f32_key(score_row)
    key = jnp.where(jnp.arange(npd) < n, key, jnp.int32(-2**31))
    kth = jnp.sort(key)[npd - k]
    gt = key > kth
    tie = key == kth
    quota = k - jnp.sum(gt.astype(jnp.int32))
    tie_rank = jnp.cumsum(tie.astype(jnp.int32))
    sel = gt | (tie & (tie_rank <= quota))
    pos = jnp.cumsum(sel.astype(jnp.int32)) - 1
    idx = jnp.where(sel, pos, -1).astype(jnp.int32)
    perm = jnp.zeros((kpd,), jnp.int32).at[
        jnp.where(sel, pos, kpd)].set(jnp.arange(npd, dtype=jnp.int32), mode="drop")
    sm = idx[src]
    dm = idx[dst]
    nm = (mask > 0) & (sm >= 0) & (dm >= 0)
    spread = (jnp.arange(src.shape[0], dtype=jnp.int32) * 37) % k
    return (perm, jnp.where(nm, sm, spread), jnp.where(nm, dm, spread),
            nm.astype(jnp.float32))


def _gather_scale_jnp(h2, score_row, perm):
    sc = score_row[perm]
    t = jnp.exp(-2.0 * jnp.abs(sc))
    tanh = jnp.sign(sc) * (1.0 - t) / (1.0 + t)
    return h2[perm] * tanh[:, None]


def _layer(x, src_b, dst_b, mask_b, n, k, W, a_s, a_d, bc, Wh, bh, pw):
    npd = x.shape[0]
    kpd = _npad(k)
    src, dst, mask = _pad_edges(src_b, dst_b, mask_b, n)
    h_chunks, als, ald = _mA(x, W, a_s, a_d)
    alphas = _edge_softmax_sc(als, ald, src, dst, mask, n)
    acc_chunks = _aggregate_sc(h_chunks, alphas, src, dst, npd)
    h2, score = _mB(acc_chunks, Wh, bc, bh, pw)
    score_row = score[0]
    perm, src_n, dst_n, mask_n = _select_jnp(score_row, n, k, kpd,
                                             src_b, dst_b, mask_b)
    xn = _gather_scale_jnp(h2, score_row, perm)
    r = _readout(xn, k)
    return xn, src_n, dst_n, mask_n, r


def kernel(x, edge_attr, edge_index, batch_index, W1, as1, ad1, bc1, Wh1, bh1,
           pw1, W2, as2, ad2, bc2, Wh2, bh2, pw2, W3, as3, ad3, bc3, Wh3, bh3,
           pw3, Wl1, bl1, Wl2, bl2):
    src = edge_index[0]
    dst = edge_index[1]
    mask = jnp.ones((E,), jnp.float32)
    x = jnp.concatenate([x, jnp.zeros((_npad(N) - N, x.shape[1]), x.dtype)])
    h, src, dst, mask, r1 = _layer(x, src, dst, mask, N, 8000,
                                   W1, as1, ad1, bc1, Wh1, bh1, pw1)
    h, src, dst, mask, r2 = _layer(h, src, dst, mask, 8000, 4000,
                                   W2, as2, ad2, bc2, Wh2, bh2, pw2)
    h, src, dst, mask, r3 = _layer(h, src, dst, mask, 4000, 800,
                                   W3, as3, ad3, bc3, Wh3, bh3, pw3)
    return _mlp(r1, r2, r3, Wl1, bl1, Wl2, bl2)


# speedup vs baseline: 5.3346x; 1.0012x over previous
import functools

import jax
import jax.numpy as jnp
import numpy as np
from jax import lax
from jax.experimental import pallas as pl
from jax.experimental.pallas import tpu as pltpu
from jax.experimental.pallas import tpu_sc as plsc

N = 10000
E = 160000
EMB = 1024
H = 3
CW = 128
NCH = (H * EMB) // CW
BM = 256

NEG_SLOPE = 0.2


def _pad_edges(src, dst, mask, n):
    ep = src.shape[0] + n
    ept = int(-(-ep // 1024) * 1024)
    loops = jnp.arange(n, dtype=jnp.int32)
    pad_idx = jnp.arange(ept - ep, dtype=jnp.int32) % n
    src = jnp.concatenate([src, loops, pad_idx])
    dst = jnp.concatenate([dst, loops, pad_idx])
    mask = jnp.concatenate([mask, jnp.ones((n,), jnp.float32),
                            jnp.zeros((ept - ep,), jnp.float32)])
    return src, dst, mask


def _mA_body(x_ref, w_ref, as_ref, ad_ref, *out_refs):
    h_refs = out_refs[:NCH]
    als_ref, ald_ref = out_refs[NCH], out_refs[NCH + 1]
    h = jnp.dot(x_ref[...], w_ref[...], preferred_element_type=jnp.float32)
    rows_s, rows_d = [], []
    for hh in range(H):
        sl = h[:, hh * EMB:(hh + 1) * EMB]
        rows_s.append(jnp.sum(sl * as_ref[hh:hh + 1, :], axis=1)[None, :])
        rows_d.append(jnp.sum(sl * ad_ref[hh:hh + 1, :], axis=1)[None, :])
    z = jnp.zeros((8 - H, x_ref.shape[0]), jnp.float32)
    als_ref[...] = jnp.concatenate(rows_s + [z], axis=0)
    ald_ref[...] = jnp.concatenate(rows_d + [z], axis=0)
    for c in range(NCH):
        h_refs[c][...] = h[:, c * CW:(c + 1) * CW]


def _mA(x, W, a_s, a_d):
    n, f = x.shape
    as_p = jnp.zeros((8, EMB), jnp.float32).at[:H].set(a_s)
    ad_p = jnp.zeros((8, EMB), jnp.float32).at[:H].set(a_d)
    grid = (n // BM,)
    out_shapes = ([jax.ShapeDtypeStruct((n, CW), jnp.float32)] * NCH
                  + [jax.ShapeDtypeStruct((8, n), jnp.float32)] * 2)
    out_specs = ([pl.BlockSpec((BM, CW), lambda m: (m, 0))] * NCH
                 + [pl.BlockSpec((8, BM), lambda m: (0, m))] * 2)
    res = pl.pallas_call(
        _mA_body,
        grid=grid,
        in_specs=[
            pl.BlockSpec((BM, f), lambda m: (m, 0)),
            pl.BlockSpec((f, H * EMB), lambda m: (0, 0)),
            pl.BlockSpec((8, EMB), lambda m: (0, 0)),
            pl.BlockSpec((8, EMB), lambda m: (0, 0)),
        ],
        out_specs=out_specs,
        out_shape=out_shapes,
    )(x, W, as_p, ad_p)
    return res[:NCH], res[NCH], res[NCH + 1]


def _mB_body(*refs):
    acc_refs = refs[:NCH]
    wh_ref, bc_ref, bh_ref, pw_ref = refs[NCH:NCH + 4]
    h2_ref, sc_ref = refs[NCH + 4], refs[NCH + 5]
    bm = acc_refs[0].shape[0]
    h2 = jnp.zeros((bm, EMB), jnp.float32)
    for c in range(NCH):
        xc = acc_refs[c][...] + bc_ref[c:c + 1, :]
        h2 = h2 + jnp.dot(xc, wh_ref[c * CW:(c + 1) * CW, :],
                          preferred_element_type=jnp.float32)
    h2 = h2 + bh_ref[0:1, :]
    h2_ref[...] = h2
    wnorm = jax.lax.rsqrt(jnp.sum(pw_ref[0:1, :] * pw_ref[0:1, :]))
    score = jnp.sum(h2 * pw_ref[0:1, :], axis=1) * wnorm
    z = jnp.zeros((7, bm), jnp.float32)
    sc_ref[...] = jnp.concatenate([score[None, :], z], axis=0)


def _mB(acc_chunks, Wh, bc, bh, pw):
    n = acc_chunks[0].shape[0]
    WhT = Wh.T
    bcp = bc.reshape(NCH, CW)
    bhp = jnp.zeros((8, EMB), jnp.float32).at[0].set(bh)
    pwp = jnp.zeros((8, EMB), jnp.float32).at[0].set(pw)
    grid = (n // BM,)
    h2, score = pl.pallas_call(
        _mB_body,
        grid=grid,
        in_specs=(
            [pl.BlockSpec((BM, CW), lambda m: (m, 0))] * NCH
            + [pl.BlockSpec((H * EMB, EMB), lambda m: (0, 0)),
               pl.BlockSpec((NCH, CW), lambda m: (0, 0)),
               pl.BlockSpec((8, EMB), lambda m: (0, 0)),
               pl.BlockSpec((8, EMB), lambda m: (0, 0))]
        ),
        out_specs=[pl.BlockSpec((BM, EMB), lambda m: (m, 0)),
                   pl.BlockSpec((8, BM), lambda m: (0, m))],
        out_shape=[jax.ShapeDtypeStruct((n, EMB), jnp.float32),
                   jax.ShapeDtypeStruct((8, n), jnp.float32)],
    )(*acc_chunks, WhT, bcp, bhp, pwp)
    return h2, score


def _readout_body(k, x_ref, out_ref):
    m = pl.program_id(0)
    rid = lax.broadcasted_iota(jnp.int32, (BM, EMB), 0) + m * BM
    valid = rid < k
    xb = x_ref[...]
    bmax = jnp.max(jnp.where(valid, xb, -1e30), axis=0)[None, :]
    bsum = (jnp.sum(jnp.where(valid, xb, 0.0), axis=0) / float(k))[None, :]
    blk = jnp.concatenate(
        [jnp.concatenate([bmax, bsum], axis=1),
         jnp.zeros((7, 2 * EMB), jnp.float32)], axis=0)

    @pl.when(m == 0)
    def _():
        out_ref[...] = blk

    @pl.when(m != 0)
    def _():
        prev = out_ref[...]
        mx = jnp.maximum(prev[0:1, :EMB], blk[0:1, :EMB])
        sm = prev[0:1, EMB:] + blk[0:1, EMB:]
        out_ref[...] = jnp.concatenate(
            [jnp.concatenate([mx, sm], axis=1), prev[1:]], axis=0)


def _readout(x, k):
    kpd = x.shape[0]
    return pl.pallas_call(
        functools.partial(_readout_body, k),
        grid=(kpd // BM,),
        in_specs=[pl.BlockSpec((BM, EMB), lambda m: (m, 0))],
        out_specs=pl.BlockSpec((8, 2 * EMB), lambda m: (0, 0)),
        out_shape=jax.ShapeDtypeStruct((8, 2 * EMB), jnp.float32),
    )(x)


def _mlp_body(r1_ref, r2_ref, r3_ref, w1_ref, b1_ref, w2_ref, b2_ref, out_ref):
    z = r1_ref[...] + r2_ref[...] + r3_ref[...]
    z = jnp.dot(z, w1_ref[...], preferred_element_type=jnp.float32) + b1_ref[0:1, :]
    z = jnp.maximum(z, 0.0)
    out_ref[...] = jnp.dot(z, w2_ref[...], preferred_element_type=jnp.float32) + b2_ref[...]


def _mlp(r1, r2, r3, Wl1, bl1, Wl2, bl2):
    W1T = Wl1.T
    W2Tp = jnp.zeros((EMB, 128), jnp.float32).at[:, :2].set(Wl2.T)
    b1p = jnp.zeros((8, EMB), jnp.float32).at[0].set(bl1)
    b2p = jnp.zeros((8, 128), jnp.float32).at[0, :2].set(bl2)
    out = pl.pallas_call(
        _mlp_body,
        in_specs=[
            pl.BlockSpec((8, 2 * EMB), lambda: (0, 0)),
            pl.BlockSpec((8, 2 * EMB), lambda: (0, 0)),
            pl.BlockSpec((8, 2 * EMB), lambda: (0, 0)),
            pl.BlockSpec((2 * EMB, EMB), lambda: (0, 0)),
            pl.BlockSpec((8, EMB), lambda: (0, 0)),
            pl.BlockSpec((EMB, 128), lambda: (0, 0)),
            pl.BlockSpec((8, 128), lambda: (0, 0)),
        ],
        out_specs=pl.BlockSpec((8, 128), lambda: (0, 0)),
        out_shape=jax.ShapeDtypeStruct((8, 128), jnp.float32),
    )(r1, r2, r3, W1T, b1p, W2Tp, b2p)
    return out[0:1, :2]


def _npad(n):
    return int(-(-n // 256) * 256)


_SC_MESH = dict(core_axis_name="c", subcore_axis_name="s")
_SC_PARAMS = pltpu.CompilerParams(needs_layout_passes=False)


def _edge_softmax_sc(als, ald, src, dst, mask, n):
    ep = src.shape[0]
    npd = _npad(n)
    ept = ep // 16
    ntile = npd // 16
    nb = ept // 16
    mesh = plsc.VectorSubcoreMesh(**_SC_MESH)

    @functools.partial(
        pl.kernel,
        out_type=[jax.ShapeDtypeStruct((ep,), jnp.float32)] * H,
        mesh=mesh,
        compiler_params=_SC_PARAMS,
        scratch_types=[
            pltpu.VMEM((npd,), jnp.float32),
            pltpu.VMEM((npd,), jnp.float32),
            pltpu.VMEM((npd,), jnp.float32),
            pltpu.VMEM((ept,), jnp.float32),
            pltpu.VMEM((ept,), jnp.int32),
            pltpu.VMEM((ept,), jnp.int32),
            pltpu.VMEM((ept,), jnp.float32),
            pltpu.VMEM((16, ntile), jnp.float32),
            pltpu.VMEM((ntile,), jnp.float32),
            pltpu.VMEM_SHARED((16, npd), jnp.float32),
            pltpu.VMEM_SHARED((npd,), jnp.float32),
        ],
    )
    def kern(als_h, ald_h, src_h, dst_h, mask_h, a0_h, a1_h, a2_h,
             als_t, ald_t, s_t, ex_t, srcv, dstv, maskv, red16, red,
             stage, merged):
        core = lax.axis_index("c")
        sub = lax.axis_index("s")
        base = sub * ept
        pltpu.sync_copy(src_h.at[pl.ds(base, ept)], srcv)
        pltpu.sync_copy(dst_h.at[pl.ds(base, ept)], dstv)
        pltpu.sync_copy(mask_h.at[pl.ds(base, ept)], maskv)
        alpha_outs = [a0_h, a1_h, a2_h]
        zero16 = jnp.zeros((16,), jnp.float32)
        for hh in range(H):
            ow = 0 if hh < 2 else 1

            @pl.when(core == ow)
            def _(hh=hh):
                a_h = alpha_outs[hh]
                pltpu.sync_copy(als_h.at[hh], als_t)
                pltpu.sync_copy(ald_h.at[hh], ald_t)

                def zb(i, _):
                    s_t[pl.ds(i * 16, 16)] = zero16
                    return 0
                lax.fori_loop(0, npd // 16, zb, 0)

                def p1(b, _):
                    sl = pl.ds(b * 16, 16)
                    d16 = dstv[sl]
                    e = (plsc.load_gather(als_t, [srcv[sl]])
                         + plsc.load_gather(ald_t, [d16]))
                    e = jnp.where(e >= 0, e, NEG_SLOPE * e)
                    exv = jnp.exp(e) * maskv[sl]
                    ex_t[sl] = exv
                    plsc.addupdate_scatter(s_t, [d16], exv)
                    return 0
                lax.fori_loop(0, nb, p1, 0)

                pltpu.sync_copy(s_t, stage.at[sub])
                plsc.subcore_barrier()
                pltpu.sync_copy(stage.at[:, pl.ds(sub * ntile, ntile)], red16)

                def p2(j, _):
                    sl = pl.ds(j * 16, 16)
                    acc = zero16
                    for r in range(16):
                        acc = acc + red16[r, sl]
                    red[sl] = acc
                    return 0
                lax.fori_loop(0, ntile // 16, p2, 0)
                pltpu.sync_copy(red, merged.at[pl.ds(sub * ntile, ntile)])
                plsc.subcore_barrier()
                pltpu.sync_copy(merged, s_t)
                plsc.subcore_barrier()

                def p3(b, _):
                    sl = pl.ds(b * 16, 16)
                    sg = plsc.load_gather(s_t, [dstv[sl]])
                    ex_t[sl] = ex_t[sl] / (sg + 1e-16)
                    return 0
                lax.fori_loop(0, nb, p3, 0)
                pltpu.sync_copy(ex_t, a_h.at[pl.ds(base, ept)])

    return list(kern(als, ald, src, dst, mask))


def _aggregate_sc(h_chunks, alphas, src, dst, npd):
    ep = src.shape[0]
    ept = ep // 16
    nb = ept // 16
    nr = npd // 16
    mesh = plsc.VectorSubcoreMesh(**_SC_MESH)
    nring = 4

    @functools.partial(
        pl.kernel,
        out_type=[jax.ShapeDtypeStruct((npd, CW), jnp.float32)] * NCH,
        mesh=mesh,
        compiler_params=_SC_PARAMS,
        scratch_types=(
            [pltpu.VMEM((ept,), jnp.int32)] * 2
            + [pltpu.VMEM((ept,), jnp.float32)]
            + [pltpu.VMEM((16, CW), jnp.float32)] * nring
            + [pltpu.VMEM((64, CW), jnp.float32)]
            + [pltpu.SemaphoreType.DMA] * nring
            + [pltpu.SemaphoreType.DMA] * nring
            + [pltpu.VMEM_SHARED((npd, CW), jnp.float32)]
        ),
    )
    def kern(*refs):
        h_refs = refs[:NCH]
        a_refs = refs[NCH:NCH + H]
        src_h, dst_h = refs[NCH + H], refs[NCH + H + 1]
        out_refs = refs[NCH + H + 2:2 * NCH + H + 2]
        scr = refs[2 * NCH + H + 2:]
        srcv, dstv, av = scr[0], scr[1], scr[2]
        bufs = scr[3:3 + nring]
        zb = scr[3 + nring]
        gsem = scr[4 + nring:4 + 2 * nring]
        ssem = scr[4 + 2 * nring:4 + 3 * nring]
        acc = scr[4 + 3 * nring]

        core = lax.axis_index("c")
        sub = lax.axis_index("s")
        base = sub * ept
        r0 = sub * nr
        pltpu.sync_copy(src_h.at[pl.ds(base, ept)], srcv)
        pltpu.sync_copy(dst_h.at[pl.ds(base, ept)], dstv)
        zero16 = jnp.zeros((16,), jnp.float32)
        for r in range(64):
            for j in range(CW // 16):
                zb[r, pl.ds(j * 16, 16)] = zero16

        def issue_gather(b, i, h_ref):
            s16 = srcv[pl.ds(b * 16, 16)]
            pltpu.async_copy(h_ref.at[s16], bufs[i], gsem[i])

        def chunk(c, h_ref, out_ref, need_alpha, a_ref):
            if need_alpha:
                pltpu.sync_copy(a_ref.at[pl.ds(base, ept)], av)

            def zloop(i, _):
                pltpu.sync_copy(zb, acc.at[pl.ds(r0 + i * 64, 64)])
                return 0
            lax.fori_loop(0, nr // 64, zloop, 0)
            plsc.subcore_barrier()

            for i in range(min(nring - 1, 3)):
                issue_gather(i, i, h_ref)

            def quad(q, _):
                for i in range(nring):
                    b = nring * q + i
                    ip = (i + nring - 1) % nring
                    @pl.when(b + nring - 1 < nb)
                    def _():
                        @pl.when(b + nring - 1 >= nring)
                        def _():
                            pltpu.make_async_copy(h_ref.at[pl.ds(0, 16)],
                                                  bufs[ip], ssem[ip]).wait()
                        issue_gather(b + nring - 1, ip, h_ref)
                    pltpu.make_async_copy(h_ref.at[pl.ds(0, 16)],
                                          bufs[i], gsem[i]).wait()

                    def row(r, _):
                        ar = plsc.load_gather(
                            av, [jnp.full((16,), b * 16 + r, jnp.int32)])
                        for j in range(CW // 16):
                            sl = pl.ds(j * 16, 16)
                            bufs[i][r, sl] = bufs[i][r, sl] * ar
                        return 0
                    lax.fori_loop(0, 16, row, 0)
                    d16 = dstv[pl.ds(b * 16, 16)]
                    pltpu.async_copy(bufs[i], acc.at[d16], ssem[i], add=True)
                return 0
            lax.fori_loop(0, nb // nring, quad, 0)
            for i in range(nring):
                pltpu.make_async_copy(h_ref.at[pl.ds(0, 16)], bufs[i],
                                      ssem[i]).wait()
            plsc.subcore_barrier()
            pltpu.sync_copy(acc.at[pl.ds(r0, nr)], out_ref.at[pl.ds(r0, nr)])

        for ow in range(2):
            @pl.when(core == ow)
            def _(ow=ow):
                last_hh = -1
                for c in range(ow * (NCH // 2), (ow + 1) * (NCH // 2)):
                    hh = c // (EMB // CW)
                    chunk(c, h_refs[c], out_refs[c], hh != last_hh,
                          a_refs[hh])
                    last_hh = hh

    return list(kern(*h_chunks, *alphas, src, dst))


def _edge_softmax_jnp(als, ald, src, dst, mask, n):
    a_s = als[:H].T
    a_d = ald[:H].T
    e = a_s[src] + a_d[dst]
    e = jnp.where(e >= 0, e, NEG_SLOPE * e)
    ex = mask[:, None] * jnp.exp(e)
    s = jax.ops.segment_sum(ex, dst, num_segments=n)
    alpha = ex / (s[dst] + 1e-16)
    return [alpha[:, hh] for hh in range(H)]


def _aggregate_jnp(h_chunks, alphas, src, dst, n):
    out = []
    for c in range(NCH):
        a = alphas[c // (EMB // CW)]
        msg = h_chunks[c][src] * a[:, None]
        out.append(jax.ops.segment_sum(msg, dst, num_segments=n))
    return out


def _f32_key(x):
    u = lax.bitcast_convert_type(x, jnp.int32)
    return jnp.where(u >= 0, u, u ^ jnp.int32(0x7FFFFFFF))


def _kth_body(n, k, sc_ref, out_ref):
    x = sc_ref[...]
    u = lax.bitcast_convert_type(x, jnp.int32)
    key = jnp.where(u >= 0, u, u ^ jnp.int32(0x7FFFFFFF))
    cols = x.shape[1]
    gid = (lax.broadcasted_iota(jnp.int32, key.shape, 0) * cols
           + lax.broadcasted_iota(jnp.int32, key.shape, 1))
    key = jnp.where(gid < n, key, jnp.int32(-2**31))
    sign = jnp.int32(-2**31)

    def step(i, t):
        cand = t | (jnp.int32(1) << (31 - i))
        cnt = jnp.sum((key >= (cand ^ sign)).astype(jnp.int32))
        return jnp.where(cnt >= k, cand, t)
    ut = lax.fori_loop(0, 32, step, jnp.int32(0))
    out_ref[...] = jnp.full((8, 128), ut ^ sign, jnp.int32)


def _kth_select(score_row, n, k):
    npd = score_row.shape[0]
    cols = npd // 8
    out = pl.pallas_call(
        functools.partial(_kth_body, n, k),
        in_specs=[pl.BlockSpec((8, cols), lambda: (0, 0))],
        out_specs=pl.BlockSpec((8, 128), lambda: (0, 0)),
        out_shape=jax.ShapeDtypeStruct((8, 128), jnp.int32),
    )(score_row.reshape(8, cols))
    return out[0, 0]


def _select_jnp(score_row, n, k, kpd, src, dst, mask):
    npd = score_row.shape[0]
    key = _f32_key(score_row)
    key = jnp.where(jnp.arange(npd) < n, key, jnp.int32(-2**31))
    kth = _kth_select(score_row, n, k)
    gt = key > kth
    tie = key == kth
    quota = k - jnp.sum(gt.astype(jnp.int32))
    tie_rank = jnp.cumsum(tie.astype(jnp.int32))
    sel = gt | (tie & (tie_rank <= quota))
    pos = jnp.cumsum(sel.astype(jnp.int32)) - 1
    idx = jnp.where(sel, pos, -1).astype(jnp.int32)
    perm = jnp.zeros((kpd,), jnp.int32).at[
        jnp.where(sel, pos, kpd)].set(jnp.arange(npd, dtype=jnp.int32), mode="drop")
    sm = idx[src]
    dm = idx[dst]
    nm = (mask > 0) & (sm >= 0) & (dm >= 0)
    spread = (jnp.arange(src.shape[0], dtype=jnp.int32) * 37) % k
    return (perm, jnp.where(nm, sm, spread), jnp.where(nm, dm, spread),
            nm.astype(jnp.float32))


def _gather_scale_jnp(h2, score_row, perm):
    sc = score_row[perm]
    t = jnp.exp(-2.0 * jnp.abs(sc))
    tanh = jnp.sign(sc) * (1.0 - t) / (1.0 + t)
    return h2[perm] * tanh[:, None]


def _layer(x, src_b, dst_b, mask_b, n, k, W, a_s, a_d, bc, Wh, bh, pw):
    npd = x.shape[0]
    kpd = _npad(k)
    src, dst, mask = _pad_edges(src_b, dst_b, mask_b, n)
    h_chunks, als, ald = _mA(x, W, a_s, a_d)
    alphas = _edge_softmax_sc(als, ald, src, dst, mask, n)
    acc_chunks = _aggregate_sc(h_chunks, alphas, src, dst, npd)
    h2, score = _mB(acc_chunks, Wh, bc, bh, pw)
    score_row = score[0]
    perm, src_n, dst_n, mask_n = _select_jnp(score_row, n, k, kpd,
                                             src_b, dst_b, mask_b)
    xn = _gather_scale_jnp(h2, score_row, perm)
    r = _readout(xn, k)
    return xn, src_n, dst_n, mask_n, r


def kernel(x, edge_attr, edge_index, batch_index, W1, as1, ad1, bc1, Wh1, bh1,
           pw1, W2, as2, ad2, bc2, Wh2, bh2, pw2, W3, as3, ad3, bc3, Wh3, bh3,
           pw3, Wl1, bl1, Wl2, bl2):
    src = edge_index[0]
    dst = edge_index[1]
    mask = jnp.ones((E,), jnp.float32)
    x = jnp.concatenate([x, jnp.zeros((_npad(N) - N, x.shape[1]), x.dtype)])
    h, src, dst, mask, r1 = _layer(x, src, dst, mask, N, 8000,
                                   W1, as1, ad1, bc1, Wh1, bh1, pw1)
    h, src, dst, mask, r2 = _layer(h, src, dst, mask, 8000, 4000,
                                   W2, as2, ad2, bc2, Wh2, bh2, pw2)
    h, src, dst, mask, r3 = _layer(h, src, dst, mask, 4000, 800,
                                   W3, as3, ad3, bc3, Wh3, bh3, pw3)
    return _mlp(r1, r2, r3, Wl1, bl1, Wl2, bl2)
